# speedup vs baseline: 2.3042x; 2.3042x over previous
import jax
import jax.numpy as jnp
from jax.experimental import pallas as pl


def _bcast_copy(table_ref, out_ref):
    blk = table_ref[...]
    out_ref[...] = jnp.broadcast_to(blk[None, :, :], out_ref.shape)


def kernel(hidden_states, position_embeddings):
    B, S, D = hidden_states.shape
    assert position_embeddings.shape == (S, D)
    BLK_S = 512
    grid = (S // BLK_S,)
    return pl.pallas_call(
        _bcast_copy,
        grid=grid,
        in_specs=[pl.BlockSpec((BLK_S, D), lambda j: (j, 0))],
        out_specs=pl.BlockSpec((B, BLK_S, D), lambda j: (0, j, 0)),
        out_shape=jax.ShapeDtypeStruct((B, S, D), position_embeddings.dtype),
    )(position_embeddings)
